# baseline (speedup 1.0000x reference)
import functools

import jax
import jax.numpy as jnp
from jax import lax
from jax.experimental import pallas as pl
from jax.experimental.pallas import tpu as pltpu
from jax.experimental.pallas import tpu_sc as plsc

NC = 2
NS = 16
SUB = 100
NSUB = 10
SUP = SUB * NSUB
ZROWS = 200


def _dotT(lhsT, rhs):
    return lax.dot_general(lhsT, rhs, (((0,), (0,)), ((), ())),
                           preferred_element_type=jnp.float32)


def _tc_compute_t(eaT_ref, wc, bc, wt2, bt2, t_ref):
    be = eaT_ref.shape[1]
    z = jnp.maximum(_dotT(eaT_ref[...], wc[...]) + bc[...], 0.0)
    t = jnp.dot(z, wt2[...], preferred_element_type=jnp.float32) + bt2[...]
    g8 = be // 8
    t_ref[0] = jnp.concatenate([t[q * g8:(q + 1) * g8, :16] for q in range(8)], axis=1)
    t_ref[1] = jnp.concatenate([t[q * g8:(q + 1) * g8, 16:] for q in range(8)], axis=1)


def _tc_update(eaT_ref, g_ref, wd, bd, wu1g, wu2, bu2, outT_ref):
    be = eaT_ref.shape[1]
    glo = jnp.concatenate([g_ref[0][:, 16 * q:16 * (q + 1)] for q in range(8)], axis=0)
    ghi = jnp.concatenate([g_ref[1][:, 16 * q:16 * (q + 1)] for q in range(8)], axis=0)
    gfull = jnp.concatenate([glo, ghi], axis=1)
    z = (_dotT(eaT_ref[...], wd[...])
         + jnp.dot(gfull, wu1g[...], preferred_element_type=jnp.float32)
         + bd[...])
    zr = jnp.maximum(z, 0.0)
    outT = lax.dot_general(wu2[...], zr, (((0,), (1,)), ((), ())),
                           preferred_element_type=jnp.float32)
    outT_ref[...] = outT + bu2[...]


def _sc_body(n_nodes, ept, t_hbm, idx_hbm, g_hbm, idx_v, rows_v, zbuf, accum, ):
    c = lax.axis_index("c")
    s = lax.axis_index("s")
    nsup = ept // SUP
    nzch = n_nodes // ZROWS

    def _zrow(i, carry):
        zbuf[i] = jnp.zeros((16,), jnp.float32)
        return carry
    lax.fori_loop(0, ZROWS, _zrow, 0)

    def _zcp(j, carry):
        ch = s + j * NS

        @pl.when(ch < nzch)
        def _():
            pltpu.sync_copy(zbuf, accum.at[pl.ds(ch * ZROWS, ZROWS)])
        return carry
    lax.fori_loop(0, (nzch + NS - 1) // NS, _zcp, 0)
    plsc.subcore_barrier()

    ebase = s * ept

    def _scatter_sup(k, carry):
        base = ebase + k * SUP
        pltpu.sync_copy(idx_hbm.at[s * nsup + k], idx_v)
        pltpu.sync_copy(t_hbm.at[c, pl.ds(base, SUP), :], rows_v)

        def _sub(j, carry2):
            pltpu.sync_copy(rows_v.at[pl.ds(j * SUB, SUB)], accum.at[idx_v.at[j]], add=True)
            return carry2
        lax.fori_loop(0, NSUB, _sub, 0)
        return carry
    lax.fori_loop(0, nsup, _scatter_sup, 0)
    plsc.subcore_barrier()

    def _gather_sup(k, carry):
        base = ebase + k * SUP
        pltpu.sync_copy(idx_hbm.at[s * nsup + k], idx_v)

        def _sub(j, carry2):
            pltpu.sync_copy(accum.at[idx_v.at[j]], rows_v.at[pl.ds(j * SUB, SUB)])
            return carry2
        lax.fori_loop(0, NSUB, _sub, 0)
        pltpu.sync_copy(rows_v, g_hbm.at[c, pl.ds(base, SUP), :])
        return carry
    lax.fori_loop(0, nsup, _gather_sup, 0)


def kernel(edge_attr, edge_index, candidate_idxs, W_vh, b_vh, W_t1, b_t1, W_t2, b_t2, W_u1, b_u1, W_u2, b_u2):
    E = edge_attr.shape[0]
    N = candidate_idxs.shape[0]
    EMB = W_vh.shape[1]
    ept = E // NS

    src = edge_index[0].astype(jnp.int32)
    eaT = edge_attr.T
    bt2 = b_t2.reshape(1, EMB)
    bu2c = b_u2.reshape(EMB, 1)
    hp = jax.lax.Precision.HIGHEST
    wc = jnp.dot(W_vh, W_t1, precision=hp)
    bc = (jnp.dot(b_vh, W_t1, precision=hp) + b_t1).reshape(1, EMB)
    wd = jnp.dot(W_vh, W_u1[:EMB], precision=hp)
    bd = (jnp.dot(b_vh, W_u1[:EMB], precision=hp) + b_u1).reshape(1, EMB)
    wu1g = W_u1[EMB:]

    BE = 6400
    grid = (E // BE,)

    def mat_spec(shape):
        return pl.BlockSpec(shape, lambda i: (0, 0))

    t2p = pl.pallas_call(
        _tc_compute_t,
        grid=grid,
        in_specs=[
            pl.BlockSpec((4, BE), lambda i: (0, i)),
            mat_spec((4, EMB)), mat_spec((1, EMB)),
            mat_spec((EMB, EMB)), mat_spec((1, EMB)),
        ],
        out_specs=pl.BlockSpec((2, BE // 8, 128), lambda i: (0, i, 0)),
        out_shape=jax.ShapeDtypeStruct((2, E // 8, 128), jnp.float32),
    )(eaT, wc, bc, W_t2, bt2)
    t2 = t2p.reshape(2, E, 16)
    idx3d = (src.reshape(E // BE, 8, BE // 8).swapaxes(1, 2)
             .reshape(E // SUP, NSUB, SUB))

    mesh = plsc.VectorSubcoreMesh(core_axis_name="c", subcore_axis_name="s",
                                  num_cores=NC, num_subcores=NS)
    g2 = pl.kernel(
        functools.partial(_sc_body, N, ept),
        mesh=mesh,
        compiler_params=pltpu.CompilerParams(use_tc_tiling_on_sc=False),
        out_type=jax.ShapeDtypeStruct((2, E, 16), jnp.float32),
        scratch_types=[
            pltpu.VMEM((NSUB, SUB), jnp.int32),
            pltpu.VMEM((SUP, 16), jnp.float32),
            pltpu.VMEM((ZROWS, 16), jnp.float32),
            pltpu.VMEM_SHARED((N, 16), jnp.float32),
        ],
    )(t2, idx3d)

    outT = pl.pallas_call(
        _tc_update,
        grid=grid,
        in_specs=[
            pl.BlockSpec((4, BE), lambda i: (0, i)),
            pl.BlockSpec((2, BE // 8, 128), lambda i: (0, i, 0)),
            mat_spec((4, EMB)), mat_spec((1, EMB)),
            mat_spec((EMB, EMB)),
            mat_spec((EMB, EMB)), mat_spec((EMB, 1)),
        ],
        out_specs=pl.BlockSpec((EMB, BE), lambda i: (0, i)),
        out_shape=jax.ShapeDtypeStruct((EMB, E), jnp.float32),
    )(eaT, g2.reshape(2, E // 8, 128), wd, bd, wu1g, W_u2, bu2c)
    return outT.T


# speedup vs baseline: 1.1093x; 1.1093x over previous
import functools

import jax
import jax.numpy as jnp
from jax import lax
from jax.experimental import pallas as pl
from jax.experimental.pallas import tpu as pltpu
from jax.experimental.pallas import tpu_sc as plsc

NC = 2
NS = 16
SUB = 125
NSUB = 10
SUP = SUB * NSUB
ZROWS = 200


def _dotT(lhsT, rhs):
    return lax.dot_general(lhsT, rhs, (((0,), (0,)), ((), ())),
                           preferred_element_type=jnp.float32)


def _tc_compute_t(eaT_ref, wc, bc, wt2, bt2, t_ref):
    be = eaT_ref.shape[1]
    z = jnp.maximum(_dotT(eaT_ref[...], wc[...]) + bc[...], 0.0)
    t = jnp.dot(z, wt2[...], preferred_element_type=jnp.float32) + bt2[...]
    g8 = be // 8
    t_ref[0] = jnp.concatenate([t[q * g8:(q + 1) * g8, :16] for q in range(8)], axis=1)
    t_ref[1] = jnp.concatenate([t[q * g8:(q + 1) * g8, 16:] for q in range(8)], axis=1)


def _tc_update(eaT_ref, g_ref, wd, bd, wu1g, wu2, bu2, outT_ref):
    be = eaT_ref.shape[1]
    glo = jnp.concatenate([g_ref[0][:, 16 * q:16 * (q + 1)] for q in range(8)], axis=0)
    ghi = jnp.concatenate([g_ref[1][:, 16 * q:16 * (q + 1)] for q in range(8)], axis=0)
    gfull = jnp.concatenate([glo, ghi], axis=1)
    z = (_dotT(eaT_ref[...], wd[...])
         + jnp.dot(gfull, wu1g[...], preferred_element_type=jnp.float32)
         + bd[...])
    zr = jnp.maximum(z, 0.0)
    outT = lax.dot_general(wu2[...], zr, (((0,), (1,)), ((), ())),
                           preferred_element_type=jnp.float32)
    outT_ref[...] = outT + bu2[...]


def _sc_body(n_nodes, ept, t_hbm, idx_hbm, g_hbm, idx_v, rows_v, zbuf, accum, lsem, ssem):
    c = lax.axis_index("c")
    s = lax.axis_index("s")
    nsup = ept // SUP
    nzch = n_nodes // ZROWS

    def _zrow(i, carry):
        zbuf[i] = jnp.zeros((16,), jnp.float32)
        return carry
    lax.fori_loop(0, ZROWS, _zrow, 0)

    def _zcp(j, carry):
        ch = s + j * NS

        @pl.when(ch < nzch)
        def _():
            pltpu.sync_copy(zbuf, accum.at[pl.ds(ch * ZROWS, ZROWS)])
        return carry
    lax.fori_loop(0, (nzch + NS - 1) // NS, _zcp, 0)
    plsc.subcore_barrier()

    ebase = s * ept

    def _scatter_sup(k, carry):
        base = ebase + k * SUP
        l1 = pltpu.async_copy(idx_hbm.at[s * nsup + k], idx_v, lsem)
        l2 = pltpu.async_copy(t_hbm.at[c, pl.ds(base, SUP), :], rows_v, lsem)
        l1.wait()
        l2.wait()
        descs = [pltpu.async_copy(rows_v.at[pl.ds(j * SUB, SUB)],
                                  accum.at[idx_v.at[j]], ssem, add=True)
                 for j in range(NSUB)]
        for d in descs:
            d.wait()
        return carry
    lax.fori_loop(0, nsup, _scatter_sup, 0)
    plsc.subcore_barrier()

    def _gather_sup(k, carry):
        base = ebase + k * SUP
        l1 = pltpu.async_copy(idx_hbm.at[s * nsup + k], idx_v, lsem)
        l1.wait()
        descs = [pltpu.async_copy(accum.at[idx_v.at[j]],
                                  rows_v.at[pl.ds(j * SUB, SUB)], ssem)
                 for j in range(NSUB)]
        for d in descs:
            d.wait()
        pltpu.sync_copy(rows_v, g_hbm.at[c, pl.ds(base, SUP), :])
        return carry
    lax.fori_loop(0, nsup, _gather_sup, 0)


def kernel(edge_attr, edge_index, candidate_idxs, W_vh, b_vh, W_t1, b_t1, W_t2, b_t2, W_u1, b_u1, W_u2, b_u2):
    E = edge_attr.shape[0]
    N = candidate_idxs.shape[0]
    EMB = W_vh.shape[1]
    ept = E // NS

    src = edge_index[0].astype(jnp.int32)
    eaT = edge_attr.T
    bt2 = b_t2.reshape(1, EMB)
    bu2c = b_u2.reshape(EMB, 1)
    hp = jax.lax.Precision.HIGHEST
    wc = jnp.dot(W_vh, W_t1, precision=hp)
    bc = (jnp.dot(b_vh, W_t1, precision=hp) + b_t1).reshape(1, EMB)
    wd = jnp.dot(W_vh, W_u1[:EMB], precision=hp)
    bd = (jnp.dot(b_vh, W_u1[:EMB], precision=hp) + b_u1).reshape(1, EMB)
    wu1g = W_u1[EMB:]

    BE = 6400
    grid = (E // BE,)

    def mat_spec(shape):
        return pl.BlockSpec(shape, lambda i: (0, 0))

    t2p = pl.pallas_call(
        _tc_compute_t,
        grid=grid,
        in_specs=[
            pl.BlockSpec((4, BE), lambda i: (0, i)),
            mat_spec((4, EMB)), mat_spec((1, EMB)),
            mat_spec((EMB, EMB)), mat_spec((1, EMB)),
        ],
        out_specs=pl.BlockSpec((2, BE // 8, 128), lambda i: (0, i, 0)),
        out_shape=jax.ShapeDtypeStruct((2, E // 8, 128), jnp.float32),
    )(eaT, wc, bc, W_t2, bt2)
    t2 = t2p.reshape(2, E, 16)
    idx3d = (src.reshape(E // BE, 8, BE // 8).swapaxes(1, 2)
             .reshape(E // SUP, NSUB, SUB))

    mesh = plsc.VectorSubcoreMesh(core_axis_name="c", subcore_axis_name="s",
                                  num_cores=NC, num_subcores=NS)
    g2 = pl.kernel(
        functools.partial(_sc_body, N, ept),
        mesh=mesh,
        compiler_params=pltpu.CompilerParams(use_tc_tiling_on_sc=False),
        out_type=jax.ShapeDtypeStruct((2, E, 16), jnp.float32),
        scratch_types=[
            pltpu.VMEM((NSUB, SUB), jnp.int32),
            pltpu.VMEM((SUP, 16), jnp.float32),
            pltpu.VMEM((ZROWS, 16), jnp.float32),
            pltpu.VMEM_SHARED((N, 16), jnp.float32),
            pltpu.SemaphoreType.DMA,
            pltpu.SemaphoreType.DMA,
        ],
    )(t2, idx3d)

    outT = pl.pallas_call(
        _tc_update,
        grid=grid,
        in_specs=[
            pl.BlockSpec((4, BE), lambda i: (0, i)),
            pl.BlockSpec((2, BE // 8, 128), lambda i: (0, i, 0)),
            mat_spec((4, EMB)), mat_spec((1, EMB)),
            mat_spec((EMB, EMB)),
            mat_spec((EMB, EMB)), mat_spec((EMB, 1)),
        ],
        out_specs=pl.BlockSpec((EMB, BE), lambda i: (0, i)),
        out_shape=jax.ShapeDtypeStruct((EMB, E), jnp.float32),
    )(eaT, g2.reshape(2, E // 8, 128), wd, bd, wu1g, W_u2, bu2c)
    return outT.T


# speedup vs baseline: 1.2127x; 1.0932x over previous
import functools

import jax
import jax.numpy as jnp
from jax import lax
from jax.experimental import pallas as pl
from jax.experimental.pallas import tpu as pltpu
from jax.experimental.pallas import tpu_sc as plsc

NC = 2
NS = 16
SUB = 125
NSUB = 10
SUP = SUB * NSUB
ZROWS = 200


def _dotT(lhsT, rhs):
    return lax.dot_general(lhsT, rhs, (((0,), (0,)), ((), ())),
                           preferred_element_type=jnp.float32)


def _tc_compute_t(eaT_ref, wc, bc, wt2, bt2, t_ref):
    be = eaT_ref.shape[1]
    z = jnp.maximum(_dotT(eaT_ref[...], wc[...]) + bc[...], 0.0)
    t = jnp.dot(z, wt2[...], preferred_element_type=jnp.float32) + bt2[...]
    g8 = be // 8
    t_ref[0] = jnp.concatenate([t[q * g8:(q + 1) * g8, :16] for q in range(8)], axis=1)
    t_ref[1] = jnp.concatenate([t[q * g8:(q + 1) * g8, 16:] for q in range(8)], axis=1)


def _tc_update(eaT_ref, g_ref, wvh, bvh, wu1h, wu1lo, wu1hi, bu1, wu2, bu2, outT_ref):
    be = eaT_ref.shape[1]
    h = _dotT(eaT_ref[...], wvh[...]) + bvh[...]
    glo = jnp.concatenate([g_ref[0][:, 16 * q:16 * (q + 1)] for q in range(8)], axis=0)
    ghi = jnp.concatenate([g_ref[1][:, 16 * q:16 * (q + 1)] for q in range(8)], axis=0)
    z = (jnp.dot(h, wu1h[...], preferred_element_type=jnp.float32)
         + jnp.dot(glo, wu1lo[...], preferred_element_type=jnp.float32)
         + jnp.dot(ghi, wu1hi[...], preferred_element_type=jnp.float32)
         + bu1[...])
    zr = jnp.maximum(z, 0.0)
    outT = lax.dot_general(wu2[...], zr, (((0,), (1,)), ((), ())),
                           preferred_element_type=jnp.float32)
    outT_ref[...] = outT + bu2[...]


def _sc_body(n_nodes, ept, t_hbm, idx_hbm, g_hbm, idx_v, rows_v, zbuf, accum, lsem, ssem):
    c = lax.axis_index("c")
    s = lax.axis_index("s")
    nsup = ept // SUP
    nzch = n_nodes // ZROWS

    def _zrow(i, carry):
        zbuf[i] = jnp.zeros((16,), jnp.float32)
        return carry
    lax.fori_loop(0, ZROWS, _zrow, 0)

    def _zcp(j, carry):
        ch = s + j * NS

        @pl.when(ch < nzch)
        def _():
            pltpu.sync_copy(zbuf, accum.at[pl.ds(ch * ZROWS, ZROWS)])
        return carry
    lax.fori_loop(0, (nzch + NS - 1) // NS, _zcp, 0)
    plsc.subcore_barrier()

    ebase = s * ept

    def _scatter_sup(k, carry):
        base = ebase + k * SUP
        l1 = pltpu.async_copy(idx_hbm.at[s * nsup + k], idx_v, lsem)
        l2 = pltpu.async_copy(t_hbm.at[c, pl.ds(base, SUP), :], rows_v, lsem)
        l1.wait()
        l2.wait()
        descs = [pltpu.async_copy(rows_v.at[pl.ds(j * SUB, SUB)],
                                  accum.at[idx_v.at[j]], ssem, add=True)
                 for j in range(NSUB)]
        for d in descs:
            d.wait()
        return carry
    lax.fori_loop(0, nsup, _scatter_sup, 0)
    plsc.subcore_barrier()

    def _gather_sup(k, carry):
        base = ebase + k * SUP
        l1 = pltpu.async_copy(idx_hbm.at[s * nsup + k], idx_v, lsem)
        l1.wait()
        descs = [pltpu.async_copy(accum.at[idx_v.at[j]],
                                  rows_v.at[pl.ds(j * SUB, SUB)], ssem)
                 for j in range(NSUB)]
        for d in descs:
            d.wait()
        pltpu.sync_copy(rows_v, g_hbm.at[c, pl.ds(base, SUP), :])
        return carry
    lax.fori_loop(0, nsup, _gather_sup, 0)


def kernel(edge_attr, edge_index, candidate_idxs, W_vh, b_vh, W_t1, b_t1, W_t2, b_t2, W_u1, b_u1, W_u2, b_u2):
    E = edge_attr.shape[0]
    N = candidate_idxs.shape[0]
    EMB = W_vh.shape[1]
    ept = E // NS

    src = edge_index[0].astype(jnp.int32)
    eaT = edge_attr.T
    bt2 = b_t2.reshape(1, EMB)
    bu2c = b_u2.reshape(EMB, 1)
    hp = jax.lax.Precision.HIGHEST
    wc = jnp.dot(W_vh, W_t1, precision=hp)
    bc = (jnp.dot(b_vh, W_t1, precision=hp) + b_t1).reshape(1, EMB)
    bvh = b_vh.reshape(1, EMB)
    bu1 = b_u1.reshape(1, EMB)
    wu1h = W_u1[:EMB]
    wu1lo = W_u1[EMB:EMB + 16]
    wu1hi = W_u1[EMB + 16:]

    BE = 6400
    grid = (E // BE,)

    def mat_spec(shape):
        return pl.BlockSpec(shape, lambda i: (0, 0))

    t2p = pl.pallas_call(
        _tc_compute_t,
        grid=grid,
        in_specs=[
            pl.BlockSpec((4, BE), lambda i: (0, i)),
            mat_spec((4, EMB)), mat_spec((1, EMB)),
            mat_spec((EMB, EMB)), mat_spec((1, EMB)),
        ],
        out_specs=pl.BlockSpec((2, BE // 8, 128), lambda i: (0, i, 0)),
        out_shape=jax.ShapeDtypeStruct((2, E // 8, 128), jnp.float32),
    )(eaT, wc, bc, W_t2, bt2)
    t2 = t2p.reshape(2, E, 16)
    idx3d = (src.reshape(E // BE, 8, BE // 8).swapaxes(1, 2)
             .reshape(E // SUP, NSUB, SUB))

    mesh = plsc.VectorSubcoreMesh(core_axis_name="c", subcore_axis_name="s",
                                  num_cores=NC, num_subcores=NS)
    g2 = pl.kernel(
        functools.partial(_sc_body, N, ept),
        mesh=mesh,
        compiler_params=pltpu.CompilerParams(use_tc_tiling_on_sc=False),
        out_type=jax.ShapeDtypeStruct((2, E, 16), jnp.float32),
        scratch_types=[
            pltpu.VMEM((NSUB, SUB), jnp.int32),
            pltpu.VMEM((SUP, 16), jnp.float32),
            pltpu.VMEM((ZROWS, 16), jnp.float32),
            pltpu.VMEM_SHARED((N, 16), jnp.float32),
            pltpu.SemaphoreType.DMA,
            pltpu.SemaphoreType.DMA,
        ],
    )(t2, idx3d)

    outT = pl.pallas_call(
        _tc_update,
        grid=grid,
        in_specs=[
            pl.BlockSpec((4, BE), lambda i: (0, i)),
            pl.BlockSpec((2, BE // 8, 128), lambda i: (0, i, 0)),
            mat_spec((4, EMB)), mat_spec((1, EMB)),
            mat_spec((EMB, EMB)), mat_spec((16, EMB)), mat_spec((16, EMB)),
            mat_spec((1, EMB)),
            mat_spec((EMB, EMB)), mat_spec((EMB, 1)),
        ],
        out_specs=pl.BlockSpec((EMB, BE), lambda i: (0, i)),
        out_shape=jax.ShapeDtypeStruct((EMB, E), jnp.float32),
    )(eaT, g2.reshape(2, E // 8, 128), W_vh, bvh, wu1h, wu1lo, wu1hi, bu1, W_u2, bu2c)
    return outT.T


# speedup vs baseline: 1.2389x; 1.0216x over previous
import functools

import jax
import jax.numpy as jnp
from jax import lax
from jax.experimental import pallas as pl
from jax.experimental.pallas import tpu as pltpu
from jax.experimental.pallas import tpu_sc as plsc

NC = 2
NS = 16
SUB = 125
NSUB = 10
SUP = SUB * NSUB
ZROWS = 200


def _dotT(lhsT, rhs):
    return lax.dot_general(lhsT, rhs, (((0,), (0,)), ((), ())),
                           preferred_element_type=jnp.float32)


def _tc_compute_t(eaT_ref, wc, bc, wt2, bt2, t_ref):
    be = eaT_ref.shape[1]
    z = jnp.maximum(_dotT(eaT_ref[...], wc[...]) + bc[...], 0.0)
    t = jnp.dot(z, wt2[...], preferred_element_type=jnp.float32) + bt2[...]
    g8 = be // 8
    t_ref[0] = jnp.concatenate([t[q * g8:(q + 1) * g8, :16] for q in range(8)], axis=1)
    t_ref[1] = jnp.concatenate([t[q * g8:(q + 1) * g8, 16:] for q in range(8)], axis=1)


def _tc_update(eaT_ref, g_ref, wvh, bvh, wu1h, wu1lo, wu1hi, bu1, wu2, bu2, outT_ref):
    be = eaT_ref.shape[1]
    h = _dotT(eaT_ref[...], wvh[...]) + bvh[...]
    glo = jnp.concatenate([g_ref[0][:, 16 * q:16 * (q + 1)] for q in range(8)], axis=0)
    ghi = jnp.concatenate([g_ref[1][:, 16 * q:16 * (q + 1)] for q in range(8)], axis=0)
    z = (jnp.dot(h, wu1h[...], preferred_element_type=jnp.float32)
         + jnp.dot(glo, wu1lo[...], preferred_element_type=jnp.float32)
         + jnp.dot(ghi, wu1hi[...], preferred_element_type=jnp.float32)
         + bu1[...])
    zr = jnp.maximum(z, 0.0)
    outT = lax.dot_general(wu2[...], zr, (((0,), (1,)), ((), ())),
                           preferred_element_type=jnp.float32)
    outT_ref[...] = outT + bu2[...]


def _sc_body(n_nodes, ept, t_hbm, idx_hbm, g_hbm, idx_v, rows_v, zbuf, accum, lsem, ssem):
    c = lax.axis_index("c")
    s = lax.axis_index("s")
    nsup = ept // SUP
    nzch = n_nodes // ZROWS

    def _zrow(i, carry):
        zbuf[i] = jnp.zeros((16,), jnp.float32)
        return carry
    lax.fori_loop(0, ZROWS, _zrow, 0)

    def _zcp(j, carry):
        ch = s + j * NS

        @pl.when(ch < nzch)
        def _():
            pltpu.sync_copy(zbuf, accum.at[pl.ds(ch * ZROWS, ZROWS)])
        return carry
    lax.fori_loop(0, (nzch + NS - 1) // NS, _zcp, 0)
    plsc.subcore_barrier()

    ebase = s * ept

    def _scatter_sup(k, carry):
        base = ebase + k * SUP
        l1 = pltpu.async_copy(idx_hbm.at[s * nsup + k], idx_v, lsem)
        l2 = pltpu.async_copy(t_hbm.at[c, pl.ds(base, SUP), :], rows_v, lsem)
        l1.wait()
        l2.wait()
        descs = [pltpu.async_copy(rows_v.at[pl.ds(j * SUB, SUB)],
                                  accum.at[idx_v.at[j]], ssem, add=True)
                 for j in range(NSUB)]
        for d in descs:
            d.wait()
        return carry
    lax.fori_loop(0, nsup, _scatter_sup, 0)
    plsc.subcore_barrier()

    def _gather_sup(k, carry):
        base = ebase + k * SUP
        l1 = pltpu.async_copy(idx_hbm.at[s * nsup + k], idx_v, lsem)
        l1.wait()
        descs = [pltpu.async_copy(accum.at[idx_v.at[j]],
                                  rows_v.at[pl.ds(j * SUB, SUB)], ssem)
                 for j in range(NSUB)]
        for d in descs:
            d.wait()
        pltpu.sync_copy(rows_v, g_hbm.at[c, pl.ds(base, SUP), :])
        return carry
    lax.fori_loop(0, nsup, _gather_sup, 0)


def kernel(edge_attr, edge_index, candidate_idxs, W_vh, b_vh, W_t1, b_t1, W_t2, b_t2, W_u1, b_u1, W_u2, b_u2):
    E = edge_attr.shape[0]
    N = candidate_idxs.shape[0]
    EMB = W_vh.shape[1]
    ept = E // NS

    src = edge_index[0].astype(jnp.int32)
    eaT = edge_attr.T
    bt2 = b_t2.reshape(1, EMB)
    bu2c = b_u2.reshape(EMB, 1)
    hp = jax.lax.Precision.HIGHEST
    wc = jnp.dot(W_vh, W_t1, precision=hp)
    bc = (jnp.dot(b_vh, W_t1, precision=hp) + b_t1).reshape(1, EMB)
    bvh = b_vh.reshape(1, EMB)
    bu1 = b_u1.reshape(1, EMB)
    wu1h = W_u1[:EMB]
    wu1lo = W_u1[EMB:EMB + 16]
    wu1hi = W_u1[EMB + 16:]

    BE = 12800
    grid = (E // BE,)

    def mat_spec(shape):
        return pl.BlockSpec(shape, lambda i: (0, 0))

    t2p = pl.pallas_call(
        _tc_compute_t,
        grid=grid,
        in_specs=[
            pl.BlockSpec((4, BE), lambda i: (0, i)),
            mat_spec((4, EMB)), mat_spec((1, EMB)),
            mat_spec((EMB, EMB)), mat_spec((1, EMB)),
        ],
        out_specs=pl.BlockSpec((2, BE // 8, 128), lambda i: (0, i, 0)),
        out_shape=jax.ShapeDtypeStruct((2, E // 8, 128), jnp.float32),
    )(eaT, wc, bc, W_t2, bt2)
    t2 = t2p.reshape(2, E, 16)
    idx3d = (src.reshape(E // BE, 8, BE // 8).swapaxes(1, 2)
             .reshape(E // SUP, NSUB, SUB))

    mesh = plsc.VectorSubcoreMesh(core_axis_name="c", subcore_axis_name="s",
                                  num_cores=NC, num_subcores=NS)
    g2 = pl.kernel(
        functools.partial(_sc_body, N, ept),
        mesh=mesh,
        compiler_params=pltpu.CompilerParams(use_tc_tiling_on_sc=False),
        out_type=jax.ShapeDtypeStruct((2, E, 16), jnp.float32),
        scratch_types=[
            pltpu.VMEM((NSUB, SUB), jnp.int32),
            pltpu.VMEM((SUP, 16), jnp.float32),
            pltpu.VMEM((ZROWS, 16), jnp.float32),
            pltpu.VMEM_SHARED((N, 16), jnp.float32),
            pltpu.SemaphoreType.DMA,
            pltpu.SemaphoreType.DMA,
        ],
    )(t2, idx3d)

    outT = pl.pallas_call(
        _tc_update,
        grid=grid,
        in_specs=[
            pl.BlockSpec((4, BE), lambda i: (0, i)),
            pl.BlockSpec((2, BE // 8, 128), lambda i: (0, i, 0)),
            mat_spec((4, EMB)), mat_spec((1, EMB)),
            mat_spec((EMB, EMB)), mat_spec((16, EMB)), mat_spec((16, EMB)),
            mat_spec((1, EMB)),
            mat_spec((EMB, EMB)), mat_spec((EMB, 1)),
        ],
        out_specs=pl.BlockSpec((EMB, BE), lambda i: (0, i)),
        out_shape=jax.ShapeDtypeStruct((EMB, E), jnp.float32),
    )(eaT, g2.reshape(2, E // 8, 128), W_vh, bvh, wu1h, wu1lo, wu1hi, bu1, W_u2, bu2c)
    return outT.T


# speedup vs baseline: 1.2519x; 1.0104x over previous
import functools

import jax
import jax.numpy as jnp
from jax import lax
from jax.experimental import pallas as pl
from jax.experimental.pallas import tpu as pltpu
from jax.experimental.pallas import tpu_sc as plsc

NC = 2
NS = 16
SUB = 125
NSUB = 10
SUP = SUB * NSUB
ZROWS = 200


def _dotT(lhsT, rhs):
    return lax.dot_general(lhsT, rhs, (((0,), (0,)), ((), ())),
                           preferred_element_type=jnp.float32)


def _tc_compute_t(eaT_ref, wc, bc, wt2, bt2, t_ref):
    be = eaT_ref.shape[1]
    z = jnp.maximum(_dotT(eaT_ref[...], wc[...]) + bc[...], 0.0)
    t = jnp.dot(z, wt2[...], preferred_element_type=jnp.float32) + bt2[...]
    g8 = be // 8
    t_ref[0] = jnp.concatenate([t[q * g8:(q + 1) * g8, :16] for q in range(8)], axis=1)
    t_ref[1] = jnp.concatenate([t[q * g8:(q + 1) * g8, 16:] for q in range(8)], axis=1)


def _tc_update(eaT_ref, g_ref, wvh, bvh, wu1h, wu1lo, wu1hi, bu1, wu2, bu2, outT_ref):
    be = eaT_ref.shape[1]
    h = _dotT(eaT_ref[...], wvh[...]) + bvh[...]
    glo = jnp.concatenate([g_ref[0][:, 16 * q:16 * (q + 1)] for q in range(8)], axis=0)
    ghi = jnp.concatenate([g_ref[1][:, 16 * q:16 * (q + 1)] for q in range(8)], axis=0)
    z = (jnp.dot(h, wu1h[...], preferred_element_type=jnp.float32)
         + jnp.dot(glo, wu1lo[...], preferred_element_type=jnp.float32)
         + jnp.dot(ghi, wu1hi[...], preferred_element_type=jnp.float32)
         + bu1[...])
    zr = jnp.maximum(z, 0.0)
    outT = lax.dot_general(wu2[...], zr, (((0,), (1,)), ((), ())),
                           preferred_element_type=jnp.float32)
    outT_ref[...] = outT + bu2[...]


def _sc_body(n_nodes, ept, t_hbm, idx_hbm, g_hbm, idx_v, rows_v, zbuf, accum, lsem, ssem):
    c = lax.axis_index("c")
    s = lax.axis_index("s")
    nsup = ept // SUP
    nzch = n_nodes // ZROWS

    def _zrow(i, carry):
        zbuf[i] = jnp.zeros((16,), jnp.float32)
        return carry
    lax.fori_loop(0, ZROWS, _zrow, 0)

    def _zcp(j, carry):
        ch = s + j * NS

        @pl.when(ch < nzch)
        def _():
            pltpu.sync_copy(zbuf, accum.at[pl.ds(ch * ZROWS, ZROWS)])
        return carry
    lax.fori_loop(0, (nzch + NS - 1) // NS, _zcp, 0)
    plsc.subcore_barrier()

    ebase = s * ept

    def _scatter_sup(k, carry):
        base = ebase + k * SUP
        l1 = pltpu.async_copy(idx_hbm.at[s * nsup + k], idx_v, lsem)
        l2 = pltpu.async_copy(t_hbm.at[c, pl.ds(base, SUP), :], rows_v, lsem)
        l1.wait()
        l2.wait()
        descs = [pltpu.async_copy(rows_v.at[pl.ds(j * SUB, SUB)],
                                  accum.at[idx_v.at[j]], ssem, add=True)
                 for j in range(NSUB)]
        for d in descs:
            d.wait()
        return carry
    lax.fori_loop(0, nsup, _scatter_sup, 0)
    plsc.subcore_barrier()

    def _gather_sup(k, carry):
        base = ebase + k * SUP
        l1 = pltpu.async_copy(idx_hbm.at[s * nsup + k], idx_v, lsem)
        l1.wait()
        descs = [pltpu.async_copy(accum.at[idx_v.at[j]],
                                  rows_v.at[pl.ds(j * SUB, SUB)], ssem)
                 for j in range(NSUB)]
        for d in descs:
            d.wait()
        pltpu.sync_copy(rows_v, g_hbm.at[c, pl.ds(base, SUP), :])
        return carry
    lax.fori_loop(0, nsup, _gather_sup, 0)


def kernel(edge_attr, edge_index, candidate_idxs, W_vh, b_vh, W_t1, b_t1, W_t2, b_t2, W_u1, b_u1, W_u2, b_u2):
    E = edge_attr.shape[0]
    N = candidate_idxs.shape[0]
    EMB = W_vh.shape[1]
    ept = E // NS

    src = edge_index[0].astype(jnp.int32)
    eaT = edge_attr.T
    bt2 = b_t2.reshape(1, EMB)
    bu2c = b_u2.reshape(EMB, 1)
    hp = jax.lax.Precision.HIGHEST
    wc = jnp.dot(W_vh, W_t1, precision=hp)
    bc = (jnp.dot(b_vh, W_t1, precision=hp) + b_t1).reshape(1, EMB)
    bvh = b_vh.reshape(1, EMB)
    bu1 = b_u1.reshape(1, EMB)
    wu1h = W_u1[:EMB]
    wu1lo = W_u1[EMB:EMB + 16]
    wu1hi = W_u1[EMB + 16:]

    BE = 32000
    grid = (E // BE,)

    def mat_spec(shape):
        return pl.BlockSpec(shape, lambda i: (0, 0))

    t2p = pl.pallas_call(
        _tc_compute_t,
        grid=grid,
        in_specs=[
            pl.BlockSpec((4, BE), lambda i: (0, i)),
            mat_spec((4, EMB)), mat_spec((1, EMB)),
            mat_spec((EMB, EMB)), mat_spec((1, EMB)),
        ],
        out_specs=pl.BlockSpec((2, BE // 8, 128), lambda i: (0, i, 0)),
        out_shape=jax.ShapeDtypeStruct((2, E // 8, 128), jnp.float32),
    )(eaT, wc, bc, W_t2, bt2)
    t2 = t2p.reshape(2, E, 16)
    idx3d = (src.reshape(E // BE, 8, BE // 8).swapaxes(1, 2)
             .reshape(E // SUP, NSUB, SUB))

    mesh = plsc.VectorSubcoreMesh(core_axis_name="c", subcore_axis_name="s",
                                  num_cores=NC, num_subcores=NS)
    g2 = pl.kernel(
        functools.partial(_sc_body, N, ept),
        mesh=mesh,
        compiler_params=pltpu.CompilerParams(use_tc_tiling_on_sc=False),
        out_type=jax.ShapeDtypeStruct((2, E, 16), jnp.float32),
        scratch_types=[
            pltpu.VMEM((NSUB, SUB), jnp.int32),
            pltpu.VMEM((SUP, 16), jnp.float32),
            pltpu.VMEM((ZROWS, 16), jnp.float32),
            pltpu.VMEM_SHARED((N, 16), jnp.float32),
            pltpu.SemaphoreType.DMA,
            pltpu.SemaphoreType.DMA,
        ],
    )(t2, idx3d)

    outT = pl.pallas_call(
        _tc_update,
        grid=grid,
        in_specs=[
            pl.BlockSpec((4, BE), lambda i: (0, i)),
            pl.BlockSpec((2, BE // 8, 128), lambda i: (0, i, 0)),
            mat_spec((4, EMB)), mat_spec((1, EMB)),
            mat_spec((EMB, EMB)), mat_spec((16, EMB)), mat_spec((16, EMB)),
            mat_spec((1, EMB)),
            mat_spec((EMB, EMB)), mat_spec((EMB, 1)),
        ],
        out_specs=pl.BlockSpec((EMB, BE), lambda i: (0, i)),
        out_shape=jax.ShapeDtypeStruct((EMB, E), jnp.float32),
    )(eaT, g2.reshape(2, E // 8, 128), W_vh, bvh, wu1h, wu1lo, wu1hi, bu1, W_u2, bu2c)
    return outT.T
